# baseline (speedup 1.0000x reference)
import functools

import jax
import jax.numpy as jnp
from jax import lax
from jax.experimental import pallas as pl
from jax.experimental.pallas import tpu as pltpu
from jax.experimental.pallas import tpu_sc as plsc

_NC, _NS, _L = 2, 16, 16
_NW = _NC * _NS
_R, _C, _K = 16384, 4096, 64
_ROWS_W = _R // _NW
_ELEMS_W = _ROWS_W * _K
_CHUNK_ROWS = 2
_CHUNK_E = _CHUNK_ROWS * _K
_N_CHUNKS = _ROWS_W // _CHUNK_ROWS

_mesh = plsc.VectorSubcoreMesh(core_axis_name="c", subcore_axis_name="s")


@functools.partial(
    pl.kernel,
    out_type=jax.ShapeDtypeStruct((_R * _K,), jnp.float32),
    mesh=_mesh,
    scratch_types=[
        pltpu.VMEM((_K,), jnp.int32),
        pltpu.VMEM((_K,), jnp.int32),
        pltpu.VMEM((_ELEMS_W,), jnp.int32),
        pltpu.VMEM((_ELEMS_W,), jnp.float32),
        pltpu.SemaphoreType.DMA,
    ],
)
def _gather_sc(in_hbm, idx_hbm, out_hbm, idx_v, colp_v, flat_v, vals_v, sem):
    wid = lax.axis_index("s") * _NC + lax.axis_index("c")
    base_row = wid * _ROWS_W
    pltpu.sync_copy(idx_hbm, idx_v)

    for v in range(_K // _L):
        vec = idx_v[pl.ds(v * _L, _L)]
        colp_v[pl.ds(v * _L, _L)] = (
            lax.shift_right_logical(vec, 7) * 1024 + lax.bitwise_and(vec, 127)
        )

    @plsc.parallel_loop(0, _N_CHUNKS, 1, unroll=2)
    def _chunk(t):
        e0 = t * _CHUNK_E
        for rr in range(_CHUNK_ROWS):
            row = base_row + t * _CHUNK_ROWS + rr
            off = (lax.shift_right_logical(row, 3) * 32768
                   + lax.bitwise_and(row, 7) * 128)
            for v in range(_K // _L):
                vec = colp_v[pl.ds(v * _L, _L)]
                flat_v[pl.ds(e0 + rr * _K + v * _L, _L)] = vec + off
        pltpu.async_copy(
            in_hbm.at[flat_v.at[pl.ds(e0, _CHUNK_E)]],
            vals_v.at[pl.ds(e0, _CHUNK_E)],
            sem,
        )

    pltpu.make_async_copy(in_hbm.at[pl.ds(0, _ELEMS_W)], vals_v, sem).wait()

    pltpu.sync_copy(vals_v, out_hbm.at[pl.ds(wid * _ELEMS_W, _ELEMS_W)])


def kernel(inputs, indices):
    tiled = inputs.reshape(_R // 8, 8, _C // 128, 128).transpose(0, 2, 1, 3)
    out = _gather_sc(tiled.reshape(-1), indices)
    return out.reshape(_R, _K)


# speedup vs baseline: 1.1000x; 1.1000x over previous
import functools

import jax
import jax.numpy as jnp
from jax import lax
from jax.experimental import pallas as pl
from jax.experimental.pallas import tpu as pltpu
from jax.experimental.pallas import tpu_sc as plsc

_NC, _NS, _L = 2, 16, 16
_NW = _NC * _NS
_R, _C, _K = 16384, 4096, 64
_ROWS_W = _R // _NW
_ELEMS_W = _ROWS_W * _K
_CHUNK_ROWS = 1
_CHUNK_E = _CHUNK_ROWS * _K
_N_CHUNKS = _ROWS_W // _CHUNK_ROWS

_mesh = plsc.VectorSubcoreMesh(core_axis_name="c", subcore_axis_name="s")


@functools.partial(
    pl.kernel,
    out_type=jax.ShapeDtypeStruct((_R, 128), jnp.float32),
    mesh=_mesh,
    scratch_types=[
        pltpu.VMEM((_K,), jnp.int32),
        pltpu.VMEM((_K,), jnp.int32),
        pltpu.VMEM((_ELEMS_W,), jnp.int32),
        pltpu.VMEM((_ROWS_W, 128), jnp.float32),
        pltpu.SemaphoreType.DMA,
    ],
)
def _gather_sc(in_hbm, idx_hbm, out_hbm, idx_v, colp_v, flat_v, vals_v, sem):
    wid = lax.axis_index("s") * _NC + lax.axis_index("c")
    base_row = wid * _ROWS_W
    pltpu.sync_copy(idx_hbm, idx_v)

    for v in range(_K // _L):
        vec = idx_v[pl.ds(v * _L, _L)]
        colp_v[pl.ds(v * _L, _L)] = (
            lax.shift_right_logical(vec, 7) * 1024 + lax.bitwise_and(vec, 127)
        )

    @pl.loop(0, _N_CHUNKS)
    def _chunk(t):
        e0 = t * _CHUNK_E
        row = base_row + t
        off = (lax.shift_right_logical(row, 3) * 32768
               + lax.bitwise_and(row, 7) * 128)
        for v in range(_K // _L):
            vec = colp_v[pl.ds(v * _L, _L)]
            flat_v[pl.ds(e0 + v * _L, _L)] = vec + off
        pltpu.async_copy(
            in_hbm.at[flat_v.at[pl.ds(e0, _CHUNK_E)]],
            vals_v.at[t, pl.ds(0, _K)],
            sem,
        )

    pltpu.make_async_copy(
        out_hbm.at[pl.ds(0, _ROWS_W // 2)], vals_v.at[pl.ds(0, _ROWS_W // 2)], sem
    ).wait()

    pltpu.sync_copy(vals_v, out_hbm.at[pl.ds(base_row, _ROWS_W)])


def kernel(inputs, indices):
    tiled = inputs.reshape(_R // 8, 8, _C // 128, 128).transpose(0, 2, 1, 3)
    out = _gather_sc(tiled.reshape(-1), indices)
    return out[:, :_K]
